# baseline (device time: 18526 ns/iter reference)
import jax
import jax.numpy as jnp
from jax import lax
from jax.experimental import pallas as pl
from jax.experimental.pallas import tpu as pltpu

N_DEV = 4
B, SQ, SKV = 2, 256, 256
HQ_LOCAL, DH = 4, 64
D_MODEL = 512
WINDOW = 128
QROWS = SQ // N_DEV


def kernel(x, Wq, K_ext, V_ext, Wo):
    my = lax.axis_index("i")
    K_loc = lax.dynamic_slice_in_dim(K_ext, my * HQ_LOCAL, HQ_LOCAL, axis=2)
    V_loc = lax.dynamic_slice_in_dim(V_ext, my * HQ_LOCAL, HQ_LOCAL, axis=2)
    K_loc = K_loc.astype(jnp.bfloat16).reshape(B, SKV, HQ_LOCAL * DH)
    V_loc = V_loc.astype(jnp.bfloat16).reshape(B, SKV, HQ_LOCAL * DH)

    def body(x_ref, wq_ref, k_ref, v_ref, wo_ref, out_ref,
             part_ref, rs_recv, ag_send, ag_recv,
             rs_send_sems, rs_recv_sems, ag_send_sems, ag_recv_sems):
        my_pos = lax.axis_index("i")

        barrier_sem = pltpu.get_barrier_semaphore()
        for k in (1, 2, 3):
            pl.semaphore_signal(
                barrier_sem, inc=1,
                device_id=(lax.rem(my_pos + k, N_DEV),),
                device_id_type=pl.DeviceIdType.MESH,
            )
        pl.semaphore_wait(barrier_sem, 3)

        qi = lax.broadcasted_iota(jnp.int32, (SQ, SKV), 0)
        ki = lax.broadcasted_iota(jnp.int32, (SQ, SKV), 1)
        mask = jnp.abs(qi - ki) <= WINDOW

        x2 = jnp.reshape(
            x_ref[...].astype(jnp.bfloat16), (B * SQ, x_ref.shape[-1])
        )
        wq = wq_ref[...].astype(jnp.bfloat16)
        wo = wo_ref[...].astype(jnp.bfloat16)
        q_all = jnp.dot(
            x2, wq, preferred_element_type=jnp.float32
        ).astype(jnp.bfloat16)
        ctx_rows = []
        for b in range(B):
            k_b = k_ref[b]
            v_b = v_ref[b]
            ctx_cols = []
            for h in range(HQ_LOCAL):
                q_bh = q_all[b * SQ:(b + 1) * SQ, h * DH:(h + 1) * DH]
                k_bh = k_b[:, h * DH:(h + 1) * DH]
                v_bh = v_b[:, h * DH:(h + 1) * DH]
                s = lax.dot_general(
                    q_bh, k_bh,
                    dimension_numbers=(((1,), (1,)), ((), ())),
                    preferred_element_type=jnp.float32,
                ) * 0.125
                s = jnp.where(mask, s, -1e9)
                s = s - jnp.max(s, axis=-1, keepdims=True)
                w = jnp.exp(s)
                w = w / jnp.sum(w, axis=-1, keepdims=True)
                ctx_bh = jnp.dot(
                    w.astype(jnp.bfloat16), v_bh,
                    preferred_element_type=jnp.float32,
                )
                ctx_cols.append(ctx_bh.astype(jnp.bfloat16))
            ctx_rows.append(jnp.concatenate(ctx_cols, axis=1))
        ctx_all = jnp.concatenate(ctx_rows, axis=0)
        part2 = jnp.dot(
            ctx_all, wo, preferred_element_type=jnp.float32
        )
        partial = jnp.reshape(part2, (B, SQ, D_MODEL))
        out_ref[...] = partial
        part_ref[...] = partial.astype(jnp.bfloat16)

        rs = {}
        for k in (1, 2, 3):
            t = lax.rem(my_pos + k, N_DEV)
            rs[k] = pltpu.make_async_remote_copy(
                src_ref=part_ref.at[:, pl.ds(t * QROWS, QROWS), :],
                dst_ref=rs_recv.at[3 - k],
                send_sem=rs_send_sems.at[k - 1],
                recv_sem=rs_recv_sems.at[3 - k],
                device_id=(t,),
                device_id_type=pl.DeviceIdType.MESH,
            )
            rs[k].start()

        acc = out_ref[:, pl.ds(my_pos * QROWS, QROWS), :]
        for k in (3, 1, 2):
            rs[k].wait_recv()
            acc = acc + rs_recv[3 - k].astype(jnp.float32)
        out_ref[:, pl.ds(my_pos * QROWS, QROWS), :] = acc
        ag_send[...] = acc.astype(jnp.bfloat16)

        ag = {}
        for k in (1, 2, 3):
            t = lax.rem(my_pos + k, N_DEV)
            ag[k] = pltpu.make_async_remote_copy(
                src_ref=ag_send,
                dst_ref=ag_recv.at[3 - k],
                send_sem=ag_send_sems.at[k - 1],
                recv_sem=ag_recv_sems.at[3 - k],
                device_id=(t,),
                device_id_type=pl.DeviceIdType.MESH,
            )
            ag[k].start()
        for k in (3, 1, 2):
            ag[k].wait_recv()
            sender = lax.rem(my_pos + (N_DEV - k), N_DEV)
            out_ref[:, pl.ds(sender * QROWS, QROWS), :] = (
                ag_recv[3 - k].astype(jnp.float32)
            )

        for k in (1, 2, 3):
            rs[k].wait_send()
            ag[k].wait_send()

    return pl.pallas_call(
        body,
        out_shape=jax.ShapeDtypeStruct((B, SQ, D_MODEL), jnp.float32),
        in_specs=[pl.BlockSpec(memory_space=pltpu.VMEM)] * 5,
        out_specs=pl.BlockSpec(memory_space=pltpu.VMEM),
        scratch_shapes=[
            pltpu.VMEM((B, SQ, D_MODEL), jnp.bfloat16),
            pltpu.VMEM((3, B, QROWS, D_MODEL), jnp.bfloat16),
            pltpu.VMEM((B, QROWS, D_MODEL), jnp.bfloat16),
            pltpu.VMEM((3, B, QROWS, D_MODEL), jnp.bfloat16),
            pltpu.SemaphoreType.DMA((3,)),
            pltpu.SemaphoreType.DMA((3,)),
            pltpu.SemaphoreType.DMA((3,)),
            pltpu.SemaphoreType.DMA((3,)),
        ],
        compiler_params=pltpu.CompilerParams(collective_id=0),
    )(x, Wq, K_loc, V_loc, Wo)


# device time: 16198 ns/iter; 1.1437x vs baseline; 1.1437x over previous
import jax
import jax.numpy as jnp
from jax import lax
from jax.experimental import pallas as pl
from jax.experimental.pallas import tpu as pltpu

N_DEV = 4
B, SQ, SKV = 2, 256, 256
HQ_LOCAL, DH = 4, 64
D_MODEL = 512
WINDOW = 128
QROWS = SQ // N_DEV


def kernel(x, Wq, K_ext, V_ext, Wo):
    my = lax.axis_index("i")
    K_loc = lax.dynamic_slice_in_dim(K_ext, my * HQ_LOCAL, HQ_LOCAL, axis=2)
    V_loc = lax.dynamic_slice_in_dim(V_ext, my * HQ_LOCAL, HQ_LOCAL, axis=2)
    K_loc = K_loc.astype(jnp.bfloat16).reshape(B, SKV, HQ_LOCAL * DH)
    V_loc = V_loc.astype(jnp.bfloat16).reshape(B, SKV, HQ_LOCAL * DH)

    def body(x_ref, wq_ref, k_ref, v_ref, wo_ref, out_ref,
             part_ref, rs_recv, ag_send, ag_recv,
             rs_send_sems, rs_recv_sems, ag_send_sems, ag_recv_sems):
        my_pos = lax.axis_index("i")

        barrier_sem = pltpu.get_barrier_semaphore()
        for k in (1, 2, 3):
            pl.semaphore_signal(
                barrier_sem, inc=1,
                device_id=(lax.rem(my_pos + k, N_DEV),),
                device_id_type=pl.DeviceIdType.MESH,
            )
        pl.semaphore_wait(barrier_sem, 3)

        qi = lax.broadcasted_iota(jnp.int32, (SQ, SKV), 0)
        ki = lax.broadcasted_iota(jnp.int32, (SQ, SKV), 1)
        bias = jnp.where(jnp.abs(qi - ki) <= WINDOW, 0.0, -1e9).astype(
            jnp.float32
        )

        x2 = jnp.reshape(
            x_ref[...].astype(jnp.bfloat16), (B * SQ, x_ref.shape[-1])
        )
        wq = wq_ref[...].astype(jnp.bfloat16)
        wo = wo_ref[...].astype(jnp.bfloat16)
        q_all = (
            jnp.dot(x2, wq, preferred_element_type=jnp.float32) * 0.125
        ).astype(jnp.bfloat16)
        ctx_rows = []
        for b in range(B):
            k_b = k_ref[b]
            v_b = v_ref[b]
            ctx_cols = []
            for h in range(HQ_LOCAL):
                q_bh = q_all[b * SQ:(b + 1) * SQ, h * DH:(h + 1) * DH]
                k_bh = k_b[:, h * DH:(h + 1) * DH]
                v_bh = v_b[:, h * DH:(h + 1) * DH]
                s = lax.dot_general(
                    q_bh, k_bh,
                    dimension_numbers=(((1,), (1,)), ((), ())),
                    preferred_element_type=jnp.float32,
                )
                w = jnp.exp(s + bias)
                inv = 1.0 / jnp.sum(w, axis=-1, keepdims=True)
                ctx_bh = jnp.dot(
                    w.astype(jnp.bfloat16), v_bh,
                    preferred_element_type=jnp.float32,
                ) * inv
                ctx_cols.append(ctx_bh.astype(jnp.bfloat16))
            ctx_rows.append(jnp.concatenate(ctx_cols, axis=1))
        ctx_all = jnp.concatenate(ctx_rows, axis=0)
        part2 = jnp.dot(
            ctx_all, wo, preferred_element_type=jnp.float32
        )
        partial = jnp.reshape(part2, (B, SQ, D_MODEL))
        part_ref[...] = partial.astype(jnp.bfloat16)

        HALF = D_MODEL // 2
        rs = {}
        for m in (0, 1):
            for k in (1, 2, 3):
                t = lax.rem(my_pos + k, N_DEV)
                rs[m, k] = pltpu.make_async_remote_copy(
                    src_ref=part_ref.at[
                        :, pl.ds(t * QROWS, QROWS), pl.ds(m * HALF, HALF)
                    ],
                    dst_ref=rs_recv.at[m, 3 - k],
                    send_sem=rs_send_sems.at[m, k - 1],
                    recv_sem=rs_recv_sems.at[m, 3 - k],
                    device_id=(t,),
                    device_id_type=pl.DeviceIdType.MESH,
                )
                rs[m, k].start()

        out_ref[...] = partial

        ag = {}
        for m in (0, 1):
            acc = out_ref[:, pl.ds(my_pos * QROWS, QROWS), pl.ds(m * HALF, HALF)]
            for k in (3, 1, 2):
                rs[m, k].wait_recv()
                acc = acc + rs_recv[m, 3 - k].astype(jnp.float32)
            out_ref[:, pl.ds(my_pos * QROWS, QROWS), pl.ds(m * HALF, HALF)] = acc
            ag_send[:, :, pl.ds(m * HALF, HALF)] = acc.astype(jnp.bfloat16)
            for k in (1, 2, 3):
                t = lax.rem(my_pos + k, N_DEV)
                ag[m, k] = pltpu.make_async_remote_copy(
                    src_ref=ag_send.at[:, :, pl.ds(m * HALF, HALF)],
                    dst_ref=ag_recv.at[m, 3 - k],
                    send_sem=ag_send_sems.at[m, k - 1],
                    recv_sem=ag_recv_sems.at[m, 3 - k],
                    device_id=(t,),
                    device_id_type=pl.DeviceIdType.MESH,
                )
                ag[m, k].start()

        for m in (0, 1):
            for k in (3, 1, 2):
                ag[m, k].wait_recv()
                sender = lax.rem(my_pos + (N_DEV - k), N_DEV)
                out_ref[
                    :, pl.ds(sender * QROWS, QROWS), pl.ds(m * HALF, HALF)
                ] = ag_recv[m, 3 - k].astype(jnp.float32)

        for m in (0, 1):
            for k in (1, 2, 3):
                rs[m, k].wait_send()
                ag[m, k].wait_send()

    return pl.pallas_call(
        body,
        out_shape=jax.ShapeDtypeStruct((B, SQ, D_MODEL), jnp.float32),
        in_specs=[pl.BlockSpec(memory_space=pltpu.VMEM)] * 5,
        out_specs=pl.BlockSpec(memory_space=pltpu.VMEM),
        scratch_shapes=[
            pltpu.VMEM((B, SQ, D_MODEL), jnp.bfloat16),
            pltpu.VMEM((2, 3, B, QROWS, D_MODEL // 2), jnp.bfloat16),
            pltpu.VMEM((B, QROWS, D_MODEL), jnp.bfloat16),
            pltpu.VMEM((2, 3, B, QROWS, D_MODEL // 2), jnp.bfloat16),
            pltpu.SemaphoreType.DMA((2, 3)),
            pltpu.SemaphoreType.DMA((2, 3)),
            pltpu.SemaphoreType.DMA((2, 3)),
            pltpu.SemaphoreType.DMA((2, 3)),
        ],
        compiler_params=pltpu.CompilerParams(collective_id=0),
    )(x, Wq, K_loc, V_loc, Wo)


# device time: 15888 ns/iter; 1.1660x vs baseline; 1.0195x over previous
import jax
import jax.numpy as jnp
from jax import lax
from jax.experimental import pallas as pl
from jax.experimental.pallas import tpu as pltpu

N_DEV = 4
B, SQ, SKV = 2, 256, 256
HQ_LOCAL, DH = 4, 64
D_MODEL = 512
WINDOW = 128
QROWS = SQ // N_DEV


def kernel(x, Wq, K_ext, V_ext, Wo):
    my = lax.axis_index("i")
    K_loc = lax.dynamic_slice_in_dim(K_ext, my * HQ_LOCAL, HQ_LOCAL, axis=2)
    V_loc = lax.dynamic_slice_in_dim(V_ext, my * HQ_LOCAL, HQ_LOCAL, axis=2)
    K_loc = K_loc.astype(jnp.bfloat16).reshape(B, SKV, HQ_LOCAL * DH)
    V_loc = V_loc.astype(jnp.bfloat16).reshape(B, SKV, HQ_LOCAL * DH)

    def body(x_ref, wq_ref, k_ref, v_ref, wo_ref, out_ref,
             part_ref, rs_recv, ag_send, ag_recv,
             rs_send_sems, rs_recv_sems, ag_send_sems, ag_recv_sems):
        my_pos = lax.axis_index("i")

        barrier_sem = pltpu.get_barrier_semaphore()
        for k in (1, 2, 3):
            pl.semaphore_signal(
                barrier_sem, inc=1,
                device_id=(lax.rem(my_pos + k, N_DEV),),
                device_id_type=pl.DeviceIdType.MESH,
            )
        pl.semaphore_wait(barrier_sem, 3)

        qi = lax.broadcasted_iota(jnp.int32, (SQ, SKV), 0)
        ki = lax.broadcasted_iota(jnp.int32, (SQ, SKV), 1)
        bias = jnp.where(jnp.abs(qi - ki) <= WINDOW, 0.0, -1e9).astype(
            jnp.float32
        )

        x2 = jnp.reshape(
            x_ref[...].astype(jnp.bfloat16), (B * SQ, x_ref.shape[-1])
        )
        wq = wq_ref[...].astype(jnp.bfloat16)
        wo = wo_ref[...].astype(jnp.bfloat16)
        q_all = (
            jnp.dot(x2, wq, preferred_element_type=jnp.float32) * 0.125
        ).astype(jnp.bfloat16)
        ctx_rows = []
        for b in range(B):
            k_b = k_ref[b]
            v_b = v_ref[b]
            ctx_cols = []
            for h in range(HQ_LOCAL):
                q_bh = q_all[b * SQ:(b + 1) * SQ, h * DH:(h + 1) * DH]
                k_bh = k_b[:, h * DH:(h + 1) * DH]
                v_bh = v_b[:, h * DH:(h + 1) * DH]
                s = lax.dot_general(
                    q_bh, k_bh,
                    dimension_numbers=(((1,), (1,)), ((), ())),
                    preferred_element_type=jnp.float32,
                )
                w = jnp.exp(s + bias)
                inv = 1.0 / jnp.sum(w, axis=-1, keepdims=True)
                ctx_bh = jnp.dot(
                    w.astype(jnp.bfloat16), v_bh,
                    preferred_element_type=jnp.float32,
                ) * inv
                ctx_cols.append(ctx_bh.astype(jnp.bfloat16))
            ctx_rows.append(jnp.concatenate(ctx_cols, axis=1))
        ctx_all = jnp.concatenate(ctx_rows, axis=0)
        part2 = jnp.dot(
            ctx_all, wo, preferred_element_type=jnp.float32
        )
        partial = jnp.reshape(part2, (B, SQ, D_MODEL))
        part_ref[...] = partial.astype(jnp.bfloat16)

        NSPLIT = 4
        HALF = D_MODEL // NSPLIT
        rs = {}
        for m in range(NSPLIT):
            for k in (1, 2, 3):
                t = lax.rem(my_pos + k, N_DEV)
                rs[m, k] = pltpu.make_async_remote_copy(
                    src_ref=part_ref.at[
                        :, pl.ds(t * QROWS, QROWS), pl.ds(m * HALF, HALF)
                    ],
                    dst_ref=rs_recv.at[m, 3 - k],
                    send_sem=rs_send_sems.at[m, k - 1],
                    recv_sem=rs_recv_sems.at[m, 3 - k],
                    device_id=(t,),
                    device_id_type=pl.DeviceIdType.MESH,
                )
                rs[m, k].start()

        out_ref[...] = partial

        ag = {}
        for m in range(NSPLIT):
            acc = out_ref[:, pl.ds(my_pos * QROWS, QROWS), pl.ds(m * HALF, HALF)]
            for k in (3, 1, 2):
                rs[m, k].wait_recv()
                acc = acc + rs_recv[m, 3 - k].astype(jnp.float32)
            out_ref[:, pl.ds(my_pos * QROWS, QROWS), pl.ds(m * HALF, HALF)] = acc
            ag_send[:, :, pl.ds(m * HALF, HALF)] = acc.astype(jnp.bfloat16)
            for k in (1, 2, 3):
                t = lax.rem(my_pos + k, N_DEV)
                ag[m, k] = pltpu.make_async_remote_copy(
                    src_ref=ag_send.at[:, :, pl.ds(m * HALF, HALF)],
                    dst_ref=ag_recv.at[m, 3 - k],
                    send_sem=ag_send_sems.at[m, k - 1],
                    recv_sem=ag_recv_sems.at[m, 3 - k],
                    device_id=(t,),
                    device_id_type=pl.DeviceIdType.MESH,
                )
                ag[m, k].start()

        for m in range(NSPLIT):
            for k in (3, 1, 2):
                ag[m, k].wait_recv()
                sender = lax.rem(my_pos + (N_DEV - k), N_DEV)
                out_ref[
                    :, pl.ds(sender * QROWS, QROWS), pl.ds(m * HALF, HALF)
                ] = ag_recv[m, 3 - k].astype(jnp.float32)

        for m in range(NSPLIT):
            for k in (1, 2, 3):
                rs[m, k].wait_send()
                ag[m, k].wait_send()

    return pl.pallas_call(
        body,
        out_shape=jax.ShapeDtypeStruct((B, SQ, D_MODEL), jnp.float32),
        in_specs=[pl.BlockSpec(memory_space=pltpu.VMEM)] * 5,
        out_specs=pl.BlockSpec(memory_space=pltpu.VMEM),
        scratch_shapes=[
            pltpu.VMEM((B, SQ, D_MODEL), jnp.bfloat16),
            pltpu.VMEM((4, 3, B, QROWS, D_MODEL // 4), jnp.bfloat16),
            pltpu.VMEM((B, QROWS, D_MODEL), jnp.bfloat16),
            pltpu.VMEM((4, 3, B, QROWS, D_MODEL // 4), jnp.bfloat16),
            pltpu.SemaphoreType.DMA((4, 3)),
            pltpu.SemaphoreType.DMA((4, 3)),
            pltpu.SemaphoreType.DMA((4, 3)),
            pltpu.SemaphoreType.DMA((4, 3)),
        ],
        compiler_params=pltpu.CompilerParams(collective_id=0),
    )(x, Wq, K_loc, V_loc, Wo)


# device time: 15878 ns/iter; 1.1668x vs baseline; 1.0006x over previous
import jax
import jax.numpy as jnp
from jax import lax
from jax.experimental import pallas as pl
from jax.experimental.pallas import tpu as pltpu

N_DEV = 4
B, SQ, SKV = 2, 256, 256
HQ_LOCAL, DH = 4, 64
D_MODEL = 512
WINDOW = 128
QROWS = SQ // N_DEV


def kernel(x, Wq, K_ext, V_ext, Wo):
    my = lax.axis_index("i")
    K_loc = lax.dynamic_slice_in_dim(K_ext, my * HQ_LOCAL, HQ_LOCAL, axis=2)
    V_loc = lax.dynamic_slice_in_dim(V_ext, my * HQ_LOCAL, HQ_LOCAL, axis=2)
    K_loc = K_loc.astype(jnp.bfloat16).reshape(B, SKV, HQ_LOCAL * DH)
    V_loc = V_loc.astype(jnp.bfloat16).reshape(B, SKV, HQ_LOCAL * DH)

    def body(x_ref, wq_ref, k_ref, v_ref, wo_ref, out_ref,
             part_ref, rs_recv, ag_send, ag_recv,
             rs_send_sems, rs_recv_sems, ag_send_sems, ag_recv_sems):
        my_pos = lax.axis_index("i")

        barrier_sem = pltpu.get_barrier_semaphore()
        for k in (1, 2, 3):
            pl.semaphore_signal(
                barrier_sem, inc=1,
                device_id=(lax.rem(my_pos + k, N_DEV),),
                device_id_type=pl.DeviceIdType.MESH,
            )
        pl.semaphore_wait(barrier_sem, 3)

        qi = lax.broadcasted_iota(jnp.int32, (SQ, SKV), 0)
        ki = lax.broadcasted_iota(jnp.int32, (SQ, SKV), 1)
        bias = jnp.where(jnp.abs(qi - ki) <= WINDOW, 0.0, -1e9).astype(
            jnp.float32
        )

        x2 = jnp.reshape(
            x_ref[...].astype(jnp.bfloat16), (B * SQ, x_ref.shape[-1])
        )
        wq = wq_ref[...].astype(jnp.bfloat16)
        wo = wo_ref[...].astype(jnp.bfloat16)
        q_all = (
            jnp.dot(x2, wq, preferred_element_type=jnp.float32) * 0.125
        ).astype(jnp.bfloat16)
        ctx_rows = []
        for b in range(B):
            k_b = k_ref[b]
            v_b = v_ref[b]
            ctx_cols = []
            for h in range(HQ_LOCAL):
                q_bh = q_all[b * SQ:(b + 1) * SQ, h * DH:(h + 1) * DH]
                k_bh = k_b[:, h * DH:(h + 1) * DH]
                v_bh = v_b[:, h * DH:(h + 1) * DH]
                s = lax.dot_general(
                    q_bh, k_bh,
                    dimension_numbers=(((1,), (1,)), ((), ())),
                    preferred_element_type=jnp.float32,
                )
                w = jnp.exp(s + bias)
                inv = 1.0 / jnp.sum(w, axis=-1, keepdims=True)
                ctx_bh = jnp.dot(
                    w.astype(jnp.bfloat16), v_bh,
                    preferred_element_type=jnp.float32,
                ) * inv
                ctx_cols.append(ctx_bh.astype(jnp.bfloat16))
            ctx_rows.append(jnp.concatenate(ctx_cols, axis=1))
        ctx_all = jnp.concatenate(ctx_rows, axis=0)
        part2 = jnp.dot(
            ctx_all, wo, preferred_element_type=jnp.float32
        )
        partial = jnp.reshape(part2, (B, SQ, D_MODEL))
        part_ref[...] = partial.astype(jnp.bfloat16)

        NSPLIT = 4
        HALF = D_MODEL // NSPLIT
        rs = {}
        for m in range(NSPLIT):
            for k in (1, 2, 3):
                t = lax.rem(my_pos + k, N_DEV)
                rs[m, k] = pltpu.make_async_remote_copy(
                    src_ref=part_ref.at[
                        :, pl.ds(t * QROWS, QROWS), pl.ds(m * HALF, HALF)
                    ],
                    dst_ref=rs_recv.at[m, 3 - k],
                    send_sem=rs_send_sems.at[m, k - 1],
                    recv_sem=rs_recv_sems.at[m, 3 - k],
                    device_id=(t,),
                    device_id_type=pl.DeviceIdType.MESH,
                )
                rs[m, k].start()

        ag = {}
        for m in range(NSPLIT):
            acc = part_ref[
                :, pl.ds(my_pos * QROWS, QROWS), pl.ds(m * HALF, HALF)
            ].astype(jnp.float32)
            for k in (3, 1, 2):
                rs[m, k].wait_recv()
                acc = acc + rs_recv[m, 3 - k].astype(jnp.float32)
            out_ref[:, pl.ds(my_pos * QROWS, QROWS), pl.ds(m * HALF, HALF)] = acc
            ag_send[:, :, pl.ds(m * HALF, HALF)] = acc.astype(jnp.bfloat16)
            for k in (1, 2, 3):
                t = lax.rem(my_pos + k, N_DEV)
                ag[m, k] = pltpu.make_async_remote_copy(
                    src_ref=ag_send.at[:, :, pl.ds(m * HALF, HALF)],
                    dst_ref=ag_recv.at[m, 3 - k],
                    send_sem=ag_send_sems.at[m, k - 1],
                    recv_sem=ag_recv_sems.at[m, 3 - k],
                    device_id=(t,),
                    device_id_type=pl.DeviceIdType.MESH,
                )
                ag[m, k].start()

        for m in range(NSPLIT):
            for k in (3, 1, 2):
                ag[m, k].wait_recv()
                sender = lax.rem(my_pos + (N_DEV - k), N_DEV)
                out_ref[
                    :, pl.ds(sender * QROWS, QROWS), pl.ds(m * HALF, HALF)
                ] = ag_recv[m, 3 - k].astype(jnp.float32)

        for m in range(NSPLIT):
            for k in (1, 2, 3):
                rs[m, k].wait_send()
                ag[m, k].wait_send()

    return pl.pallas_call(
        body,
        out_shape=jax.ShapeDtypeStruct((B, SQ, D_MODEL), jnp.float32),
        in_specs=[pl.BlockSpec(memory_space=pltpu.VMEM)] * 5,
        out_specs=pl.BlockSpec(memory_space=pltpu.VMEM),
        scratch_shapes=[
            pltpu.VMEM((B, SQ, D_MODEL), jnp.bfloat16),
            pltpu.VMEM((4, 3, B, QROWS, D_MODEL // 4), jnp.bfloat16),
            pltpu.VMEM((B, QROWS, D_MODEL), jnp.bfloat16),
            pltpu.VMEM((4, 3, B, QROWS, D_MODEL // 4), jnp.bfloat16),
            pltpu.SemaphoreType.DMA((4, 3)),
            pltpu.SemaphoreType.DMA((4, 3)),
            pltpu.SemaphoreType.DMA((4, 3)),
            pltpu.SemaphoreType.DMA((4, 3)),
        ],
        compiler_params=pltpu.CompilerParams(collective_id=0),
    )(x, Wq, K_loc, V_loc, Wo)
